# baseline (speedup 1.0000x reference)
import functools

import jax
import jax.numpy as jnp
from jax import lax
from jax.experimental import pallas as pl
from jax.experimental.pallas import tpu as pltpu
from jax.experimental.pallas import tpu_sc as plsc

L_GRID = 8192
C = 64
N = 524288
LANES = 16
NW = 32
NQ = N // NW
G = 128
NCH = NQ // G
GROUPS = G // LANES
GP = G + 3


def _spline_body(t_hbm, coeffs_hbm, out_hbm, t_v, idx_v,
                 rows_v0, rows_v1, out_v0, out_v1,
                 gsem0, gsem1, osem0, osem1):
    rows_b = (rows_v0, rows_v1)
    out_b = (out_v0, out_v1)
    gsem = (gsem0, gsem1)
    osem = (osem0, osem1)
    wid = lax.axis_index("s") * 2 + lax.axis_index("c")
    base = wid * NQ

    pltpu.sync_copy(t_hbm.at[pl.ds(base, NQ)], t_v)

    def idx_body(g, _):
        tv = t_v[pl.ds(g * LANES, LANES)]
        ti = tv.astype(jnp.int32)
        tf = ti.astype(jnp.float32)
        ceil_m1 = ti + jnp.where(tv > tf, 0, -1)
        idx = jnp.minimum(jnp.maximum(ceil_m1, 0), L_GRID - 1)
        frac = tv - idx.astype(jnp.float32)
        row = g // GROUPS
        col = (g % GROUPS) * LANES
        idx_v[row, pl.ds(col, LANES)] = idx
        t_v[pl.ds(g * LANES, LANES)] = frac
        return 0

    lax.fori_loop(0, NQ // LANES, idx_body, 0)

    chan_iota = lax.iota(jnp.int32, LANES)
    cvs = [s * LANES + chan_iota for s in range(C // LANES)]

    def compute_chunk(j, rows_v, out_v):
        @plsc.parallel_loop(0, GROUPS, unroll=2)
        def g_body(g):
            frac16 = t_v[pl.ds(j * G + g * LANES, LANES)]
            for i in range(LANES):
                q = g * LANES + i
                qv = jnp.full((LANES,), 0, jnp.int32) + q
                frac = jnp.full((LANES,), frac16[i])
                f3 = frac * (1.0 / 3.0)
                for p in range(2):
                    ws = [rows_v[q, pl.ds((2 * gr + p) * LANES, LANES)]
                          for gr in range(4)]
                    lo = [plsc.bitcast(w << 16, jnp.float32) for w in ws]
                    hi = [plsc.bitcast(w & jnp.int32(-65536), jnp.float32)
                          for w in ws]
                    for which, (a, b, cc, dd) in ((0, lo), (1, hi)):
                        s = 2 * p + which
                        inner = 0.5 * cc + dd * f3
                        inner = b + inner * frac
                        plsc.store_scatter(out_v, [cvs[s], qv],
                                           a + inner * frac)

    pltpu.async_copy(coeffs_hbm.at[idx_v.at[0]], rows_b[0], gsem[0])

    def pair_body(jj, _):
        for b in range(2):
            j = jj * 2 + b

            @pl.when(j + 1 < NCH)
            def _():
                pltpu.async_copy(coeffs_hbm.at[idx_v.at[j + 1]],
                                 rows_b[1 - b], gsem[1 - b])

            pltpu.make_async_copy(coeffs_hbm.at[idx_v.at[j]],
                                  rows_b[b], gsem[b]).wait()

            @pl.when(j >= 2)
            def _():
                pltpu.make_async_copy(
                    out_b[b].at[:, pl.ds(0, G)],
                    out_hbm.at[:, pl.ds(base + (j - 2) * G, G)],
                    osem[b]).wait()

            compute_chunk(j, rows_b[b], out_b[b])
            pltpu.async_copy(out_b[b].at[:, pl.ds(0, G)],
                             out_hbm.at[:, pl.ds(base + j * G, G)],
                             osem[b])
        return 0

    lax.fori_loop(0, NCH // 2, pair_body, 0)

    for b in range(2):
        pltpu.make_async_copy(
            out_b[b].at[:, pl.ds(0, G)],
            out_hbm.at[:, pl.ds(base + (NCH - 2 + b) * G, G)],
            osem[b]).wait()


@jax.jit
def _spline_sc(t, packed):
    mesh = plsc.VectorSubcoreMesh(core_axis_name="c", subcore_axis_name="s")
    return pl.kernel(
        _spline_body,
        mesh=mesh,
        compiler_params=pltpu.CompilerParams(needs_layout_passes=False),
        out_type=jax.ShapeDtypeStruct((C, N), jnp.float32),
        scratch_types=[
            pltpu.VMEM((NQ,), jnp.float32),
            pltpu.VMEM((NCH, G), jnp.int32),
            pltpu.VMEM((G, 2 * C), jnp.int32),
            pltpu.VMEM((G, 2 * C), jnp.int32),
            pltpu.VMEM((C, GP), jnp.float32),
            pltpu.VMEM((C, GP), jnp.float32),
            pltpu.SemaphoreType.DMA,
            pltpu.SemaphoreType.DMA,
            pltpu.SemaphoreType.DMA,
            pltpu.SemaphoreType.DMA,
        ],
    )(t, packed)


def kernel(t, coeffs, t_grid):
    del t_grid
    cb = coeffs.astype(jnp.bfloat16)
    x = cb.reshape(L_GRID, 8, 2, LANES).transpose(0, 1, 3, 2)
    u16 = lax.bitcast_convert_type(x, jnp.uint16)
    packed = lax.bitcast_convert_type(u16, jnp.int32).reshape(L_GRID, 2 * C)
    return _spline_sc(t, packed).T


# speedup vs baseline: 1.3917x; 1.3917x over previous
import functools

import jax
import jax.numpy as jnp
from jax import lax
from jax.experimental import pallas as pl
from jax.experimental.pallas import tpu as pltpu
from jax.experimental.pallas import tpu_sc as plsc

L_GRID = 8192
C = 64
N = 524288
LANES = 16
NW = 32
K = 2
NK = N // K
NQ = NK // NW
G = 128
NCH = NQ // G
GROUPS = G // LANES


def _spline_body(t_hbm, coeffs_hbm, out_hbm, t_v, idx_v,
                 rows_v0, rows_v1, out_v0, out_v1,
                 gsem0, gsem1, osem0, osem1):
    rows_b = (rows_v0, rows_v1)
    out_b = (out_v0, out_v1)
    gsem = (gsem0, gsem1)
    osem = (osem0, osem1)
    wid = lax.axis_index("s") * 2 + lax.axis_index("c")
    base = wid * NQ

    pltpu.sync_copy(t_hbm.at[pl.ds(base, NQ)], t_v)

    def idx_body(g, _):
        tv = t_v[pl.ds(g * LANES, LANES)]
        ti = tv.astype(jnp.int32)
        tf = ti.astype(jnp.float32)
        ceil_m1 = ti + jnp.where(tv > tf, 0, -1)
        idx = jnp.minimum(jnp.maximum(ceil_m1, 0), L_GRID - 1)
        frac = tv - idx.astype(jnp.float32)
        row = g // GROUPS
        col = (g % GROUPS) * LANES
        idx_v[row, pl.ds(col, LANES)] = idx
        t_v[pl.ds(g * LANES, LANES)] = frac
        return 0

    lax.fori_loop(0, NQ // LANES, idx_body, 0)

    def compute_chunk(j, rows_v, out_v):
        @plsc.parallel_loop(0, GROUPS, unroll=2)
        def g_body(g):
            frac16 = t_v[pl.ds(j * G + g * LANES, LANES)]
            for i in range(LANES):
                q = g * LANES + i
                frac = jnp.full((LANES,), frac16[i])
                f3 = frac * (1.0 / 3.0)
                for p in range(2):
                    ws = [rows_v[q, pl.ds((2 * gr + p) * LANES, LANES)]
                          for gr in range(4)]
                    lo = [plsc.bitcast(w << 16, jnp.float32) for w in ws]
                    hi = [plsc.bitcast(w & jnp.int32(-65536), jnp.float32)
                          for w in ws]
                    for which, (a, b, cc, dd) in ((0, lo), (1, hi)):
                        s = 2 * p + which
                        inner = 0.5 * cc + dd * f3
                        inner = b + inner * frac
                        out_v[q, pl.ds(s * LANES, LANES)] = a + inner * frac

    pltpu.async_copy(coeffs_hbm.at[idx_v.at[0]], rows_b[0], gsem[0])

    def pair_body(jj, _):
        for b in range(2):
            j = jj * 2 + b

            @pl.when(j + 1 < NCH)
            def _():
                pltpu.async_copy(coeffs_hbm.at[idx_v.at[j + 1]],
                                 rows_b[1 - b], gsem[1 - b])

            pltpu.make_async_copy(coeffs_hbm.at[idx_v.at[j]],
                                  rows_b[b], gsem[b]).wait()

            @pl.when(j >= 2)
            def _():
                pltpu.make_async_copy(
                    out_b[b], out_hbm.at[pl.ds(base + (j - 2) * G, G)],
                    osem[b]).wait()

            compute_chunk(j, rows_b[b], out_b[b])
            pltpu.async_copy(out_b[b], out_hbm.at[pl.ds(base + j * G, G)],
                             osem[b])
        return 0

    lax.fori_loop(0, NCH // 2, pair_body, 0)

    for b in range(2):
        pltpu.make_async_copy(
            out_b[b], out_hbm.at[pl.ds(base + (NCH - 2 + b) * G, G)],
            osem[b]).wait()


@jax.jit
def _spline_sc(t_part, packed):
    mesh = plsc.VectorSubcoreMesh(core_axis_name="c", subcore_axis_name="s")
    return pl.kernel(
        _spline_body,
        mesh=mesh,
        compiler_params=pltpu.CompilerParams(needs_layout_passes=False),
        out_type=jax.ShapeDtypeStruct((NK, C), jnp.float32),
        scratch_types=[
            pltpu.VMEM((NQ,), jnp.float32),
            pltpu.VMEM((NCH, G), jnp.int32),
            pltpu.VMEM((G, 2 * C), jnp.int32),
            pltpu.VMEM((G, 2 * C), jnp.int32),
            pltpu.VMEM((G, C), jnp.float32),
            pltpu.VMEM((G, C), jnp.float32),
            pltpu.SemaphoreType.DMA,
            pltpu.SemaphoreType.DMA,
            pltpu.SemaphoreType.DMA,
            pltpu.SemaphoreType.DMA,
        ],
    )(t_part, packed)


def kernel(t, coeffs, t_grid):
    del t_grid
    cb = coeffs.astype(jnp.bfloat16)
    x = cb.reshape(L_GRID, 8, 2, LANES).transpose(0, 1, 3, 2)
    u16 = lax.bitcast_convert_type(x, jnp.uint16)
    packed = lax.bitcast_convert_type(u16, jnp.int32).reshape(L_GRID, 2 * C)
    parts = [_spline_sc(lax.slice(t, (i * NK,), ((i + 1) * NK,)), packed)
             for i in range(K)]
    return jnp.concatenate(parts, axis=0) if K > 1 else parts[0]


# speedup vs baseline: 1.8161x; 1.3049x over previous
import functools

import jax
import jax.numpy as jnp
from jax import lax
from jax.experimental import pallas as pl
from jax.experimental.pallas import tpu as pltpu
from jax.experimental.pallas import tpu_sc as plsc

L_GRID = 8192
C = 64
N = 524288
LANES = 16
NW = 32
K = 1
NK = N // K
NQ = NK // NW
G = 128
NCH = NQ // G
GROUPS = G // LANES


def _spline_body(t_hbm, coeffs_hbm, out_hbm, t_v, idx_v,
                 rows_v0, rows_v1, out_v0, out_v1,
                 gsem0, gsem1, osem0, osem1):
    rows_b = (rows_v0, rows_v1)
    out_b = (out_v0, out_v1)
    gsem = (gsem0, gsem1)
    osem = (osem0, osem1)
    wid = lax.axis_index("s") * 2 + lax.axis_index("c")
    base = wid * NQ

    pltpu.sync_copy(t_hbm.at[pl.ds(base, NQ)], t_v)

    def idx_body(g, _):
        tv = t_v[pl.ds(g * LANES, LANES)]
        ti = tv.astype(jnp.int32)
        tf = ti.astype(jnp.float32)
        ceil_m1 = ti + jnp.where(tv > tf, 0, -1)
        idx = jnp.minimum(jnp.maximum(ceil_m1, 0), L_GRID - 1)
        frac = tv - idx.astype(jnp.float32)
        row = g // GROUPS
        col = (g % GROUPS) * LANES
        idx_v[row, pl.ds(col, LANES)] = idx
        t_v[pl.ds(g * LANES, LANES)] = frac
        return 0

    lax.fori_loop(0, NQ // LANES, idx_body, 0)

    lane_iota = lax.iota(jnp.int32, LANES)

    def transpose16(vs):
        cur = list(vs)
        for d in (1, 2, 4, 8):
            msk = (lane_iota & d) == 0
            pm = (lane_iota - d) & (LANES - 1)
            pp = (lane_iota + d) & (LANES - 1)
            nxt = []
            for i in range(LANES):
                part = cur[i ^ d]
                if (i & d) == 0:
                    sh = jnp.take_along_axis(part, pm, axis=0)
                    nxt.append(jnp.where(msk, cur[i], sh))
                else:
                    sh = jnp.take_along_axis(part, pp, axis=0)
                    nxt.append(jnp.where(msk, sh, cur[i]))
            cur = nxt
        return cur

    def compute_chunk(j, rows_v, out_v):
        @plsc.parallel_loop(0, GROUPS, unroll=2)
        def g_body(g):
            frac16 = t_v[pl.ds(j * G + g * LANES, LANES)]
            for p in range(2):
                res = ([], [])
                for i in range(LANES):
                    q = g * LANES + i
                    frac = jnp.full((LANES,), frac16[i])
                    f3 = frac * (1.0 / 3.0)
                    ws = [rows_v[q, pl.ds((2 * gr + p) * LANES, LANES)]
                          for gr in range(4)]
                    lo = [plsc.bitcast(w << 16, jnp.float32) for w in ws]
                    hi = [plsc.bitcast(w & jnp.int32(-65536), jnp.float32)
                          for w in ws]
                    for which, (a, b, cc, dd) in ((0, lo), (1, hi)):
                        inner = 0.5 * cc + dd * f3
                        inner = b + inner * frac
                        res[which].append(a + inner * frac)
                for which in range(2):
                    s = 2 * p + which
                    tr = transpose16(res[which])
                    for r in range(LANES):
                        out_v[s * LANES + r, pl.ds(g * LANES, LANES)] = tr[r]

    pltpu.async_copy(coeffs_hbm.at[idx_v.at[0]], rows_b[0], gsem[0])

    def pair_body(jj, _):
        for b in range(2):
            j = jj * 2 + b

            @pl.when(j + 1 < NCH)
            def _():
                pltpu.async_copy(coeffs_hbm.at[idx_v.at[j + 1]],
                                 rows_b[1 - b], gsem[1 - b])

            pltpu.make_async_copy(coeffs_hbm.at[idx_v.at[j]],
                                  rows_b[b], gsem[b]).wait()

            @pl.when(j >= 2)
            def _():
                pltpu.make_async_copy(
                    out_b[b], out_hbm.at[:, pl.ds(base + (j - 2) * G, G)],
                    osem[b]).wait()

            compute_chunk(j, rows_b[b], out_b[b])
            pltpu.async_copy(out_b[b], out_hbm.at[:, pl.ds(base + j * G, G)],
                             osem[b])
        return 0

    lax.fori_loop(0, NCH // 2, pair_body, 0)

    for b in range(2):
        pltpu.make_async_copy(
            out_b[b], out_hbm.at[:, pl.ds(base + (NCH - 2 + b) * G, G)],
            osem[b]).wait()


@jax.jit
def _spline_sc(t_part, packed):
    mesh = plsc.VectorSubcoreMesh(core_axis_name="c", subcore_axis_name="s")
    return pl.kernel(
        _spline_body,
        mesh=mesh,
        compiler_params=pltpu.CompilerParams(needs_layout_passes=False),
        out_type=jax.ShapeDtypeStruct((C, NK), jnp.float32),
        scratch_types=[
            pltpu.VMEM((NQ,), jnp.float32),
            pltpu.VMEM((NCH, G), jnp.int32),
            pltpu.VMEM((G, 2 * C), jnp.int32),
            pltpu.VMEM((G, 2 * C), jnp.int32),
            pltpu.VMEM((C, G), jnp.float32),
            pltpu.VMEM((C, G), jnp.float32),
            pltpu.SemaphoreType.DMA,
            pltpu.SemaphoreType.DMA,
            pltpu.SemaphoreType.DMA,
            pltpu.SemaphoreType.DMA,
        ],
    )(t_part, packed)


def kernel(t, coeffs, t_grid):
    del t_grid
    cb = coeffs.astype(jnp.bfloat16)
    x = cb.reshape(L_GRID, 8, 2, LANES).transpose(0, 1, 3, 2)
    u16 = lax.bitcast_convert_type(x, jnp.uint16)
    packed = lax.bitcast_convert_type(u16, jnp.int32).reshape(L_GRID, 2 * C)
    return _spline_sc(t, packed).T


# speedup vs baseline: 2.4637x; 1.3566x over previous
import functools

import jax
import jax.numpy as jnp
from jax import lax
from jax.experimental import pallas as pl
from jax.experimental.pallas import tpu as pltpu
from jax.experimental.pallas import tpu_sc as plsc

L_GRID = 8192
C = 64
N = 524288
LANES = 16
NW = 32
K = 1
NK = N // K
NQ = NK // NW
G = 128
NCH = NQ // G
GROUPS = G // LANES


def _spline_body(t_hbm, coeffs_hbm, out_hbm, t_v, idx_v,
                 rows_v0, rows_v1, out_v0, out_v1,
                 gsem0, gsem1, osem0, osem1):
    rows_b = (rows_v0, rows_v1)
    out_b = (out_v0, out_v1)
    gsem = (gsem0, gsem1)
    osem = (osem0, osem1)
    wid = lax.axis_index("s") * 2 + lax.axis_index("c")
    base = wid * NQ

    pltpu.sync_copy(t_hbm.at[pl.ds(base, NQ)], t_v)

    def idx_body(g, _):
        tv = t_v[pl.ds(g * LANES, LANES)]
        ti = tv.astype(jnp.int32)
        tf = ti.astype(jnp.float32)
        ceil_m1 = ti + jnp.where(tv > tf, 0, -1)
        idx = jnp.minimum(jnp.maximum(ceil_m1, 0), L_GRID - 1)
        frac = tv - idx.astype(jnp.float32)
        row = g // GROUPS
        col = (g % GROUPS) * LANES
        idx_v[row, pl.ds(col, LANES)] = idx
        t_v[pl.ds(g * LANES, LANES)] = frac
        return 0

    lax.fori_loop(0, NQ // LANES, idx_body, 0)

    lane_iota = lax.iota(jnp.int32, LANES)

    def transpose16(vs):
        cur = list(vs)
        for d in (1, 2, 4, 8):
            msk = (lane_iota & d) == 0
            pm = (lane_iota - d) & (LANES - 1)
            pp = (lane_iota + d) & (LANES - 1)
            nxt = []
            for i in range(LANES):
                part = cur[i ^ d]
                if (i & d) == 0:
                    sh = jnp.take_along_axis(part, pm, axis=0)
                    nxt.append(jnp.where(msk, cur[i], sh))
                else:
                    sh = jnp.take_along_axis(part, pp, axis=0)
                    nxt.append(jnp.where(msk, sh, cur[i]))
            cur = nxt
        return cur

    def compute_chunk(j, rows_v, out_v):
        @plsc.parallel_loop(0, GROUPS, unroll=1)
        def g_body(g):
            frac16 = t_v[pl.ds(j * G + g * LANES, LANES)]
            for p in range(2):
                res = ([], [])
                for i in range(LANES):
                    q = g * LANES + i
                    frac = jnp.full((LANES,), frac16[i])
                    f3 = frac * (1.0 / 3.0)
                    ws = [rows_v[q, pl.ds((2 * gr + p) * LANES, LANES)]
                          for gr in range(4)]
                    lo = [plsc.bitcast(w << 16, jnp.float32) for w in ws]
                    hi = [plsc.bitcast(w & jnp.int32(-65536), jnp.float32)
                          for w in ws]
                    for which, (a, b, cc, dd) in ((0, lo), (1, hi)):
                        inner = 0.5 * cc + dd * f3
                        inner = b + inner * frac
                        res[which].append(a + inner * frac)
                for which in range(2):
                    s = 2 * p + which
                    tr = transpose16(res[which])
                    for r in range(LANES):
                        out_v[s * LANES + r, pl.ds(g * LANES, LANES)] = tr[r]

    pltpu.async_copy(coeffs_hbm.at[idx_v.at[0]], rows_b[0], gsem[0])

    def pair_body(jj, _):
        for b in range(2):
            j = jj * 2 + b

            @pl.when(j + 1 < NCH)
            def _():
                pltpu.async_copy(coeffs_hbm.at[idx_v.at[j + 1]],
                                 rows_b[1 - b], gsem[1 - b])

            pltpu.make_async_copy(coeffs_hbm.at[idx_v.at[j]],
                                  rows_b[b], gsem[b]).wait()

            @pl.when(j >= 2)
            def _():
                pltpu.make_async_copy(
                    out_b[b], out_hbm.at[:, pl.ds(base + (j - 2) * G, G)],
                    osem[b]).wait()

            compute_chunk(j, rows_b[b], out_b[b])
            pltpu.async_copy(out_b[b], out_hbm.at[:, pl.ds(base + j * G, G)],
                             osem[b])
        return 0

    lax.fori_loop(0, NCH // 2, pair_body, 0)

    for b in range(2):
        pltpu.make_async_copy(
            out_b[b], out_hbm.at[:, pl.ds(base + (NCH - 2 + b) * G, G)],
            osem[b]).wait()


@jax.jit
def _spline_sc(t_part, packed):
    mesh = plsc.VectorSubcoreMesh(core_axis_name="c", subcore_axis_name="s")
    return pl.kernel(
        _spline_body,
        mesh=mesh,
        compiler_params=pltpu.CompilerParams(needs_layout_passes=False),
        out_type=jax.ShapeDtypeStruct((C, NK), jnp.float32),
        scratch_types=[
            pltpu.VMEM((NQ,), jnp.float32),
            pltpu.VMEM((NCH, G), jnp.int32),
            pltpu.VMEM((G, 2 * C), jnp.int32),
            pltpu.VMEM((G, 2 * C), jnp.int32),
            pltpu.VMEM((C, G), jnp.float32),
            pltpu.VMEM((C, G), jnp.float32),
            pltpu.SemaphoreType.DMA,
            pltpu.SemaphoreType.DMA,
            pltpu.SemaphoreType.DMA,
            pltpu.SemaphoreType.DMA,
        ],
    )(t_part, packed)


def kernel(t, coeffs, t_grid):
    del t_grid
    cb = coeffs.astype(jnp.bfloat16)
    x = cb.reshape(L_GRID, 8, 2, LANES).transpose(0, 1, 3, 2)
    u16 = lax.bitcast_convert_type(x, jnp.uint16)
    packed = lax.bitcast_convert_type(u16, jnp.int32).reshape(L_GRID, 2 * C)
    return _spline_sc(t, packed).T


# speedup vs baseline: 2.5312x; 1.0274x over previous
import functools

import jax
import jax.numpy as jnp
from jax import lax
from jax.experimental import pallas as pl
from jax.experimental.pallas import tpu as pltpu
from jax.experimental.pallas import tpu_sc as plsc

L_GRID = 8192
C = 64
N = 524288
LANES = 16
NW = 32
K = 1
NK = N // K
NQ = NK // NW
G = 128
NCH = NQ // G
GROUPS = G // LANES


def _spline_body(t_hbm, coeffs_hbm, out_hbm, t_v, idx_v,
                 rows_v0, rows_v1, out_v0, out_v1,
                 gsem0, gsem1, osem0, osem1):
    rows_b = (rows_v0, rows_v1)
    out_b = (out_v0, out_v1)
    gsem = (gsem0, gsem1)
    osem = (osem0, osem1)
    wid = lax.axis_index("s") * 2 + lax.axis_index("c")
    base = wid * NQ

    pltpu.sync_copy(t_hbm.at[pl.ds(base, NQ)], t_v)

    def idx_body(g, _):
        tv = t_v[pl.ds(g * LANES, LANES)]
        ti = tv.astype(jnp.int32)
        tf = ti.astype(jnp.float32)
        ceil_m1 = ti + jnp.where(tv > tf, 0, -1)
        idx = jnp.minimum(jnp.maximum(ceil_m1, 0), L_GRID - 1)
        frac = tv - idx.astype(jnp.float32)
        row = g // GROUPS
        col = (g % GROUPS) * LANES
        idx_v[row, pl.ds(col, LANES)] = idx
        t_v[pl.ds(g * LANES, LANES)] = frac
        return 0

    lax.fori_loop(0, NQ // LANES, idx_body, 0)

    lane_iota = lax.iota(jnp.int32, LANES)

    def transpose16(vs):
        cur = list(vs)
        for d in (1, 2, 4, 8):
            msk = (lane_iota & d) == 0
            pm = (lane_iota - d) & (LANES - 1)
            pp = (lane_iota + d) & (LANES - 1)
            nxt = []
            for i in range(LANES):
                part = cur[i ^ d]
                if (i & d) == 0:
                    sh = jnp.take_along_axis(part, pm, axis=0)
                    nxt.append(jnp.where(msk, cur[i], sh))
                else:
                    sh = jnp.take_along_axis(part, pp, axis=0)
                    nxt.append(jnp.where(msk, sh, cur[i]))
            cur = nxt
        return cur

    def compute_chunk(j, rows_v, out_v):
        @plsc.parallel_loop(0, GROUPS, unroll=1)
        def g_body(g):
            frac16 = t_v[pl.ds(j * G + g * LANES, LANES)]
            for p in range(2):
                res = ([], [])
                for i in range(LANES):
                    q = g * LANES + i
                    frac = jnp.full((LANES,), frac16[i])
                    ws = [rows_v[q, pl.ds((2 * gr + p) * LANES, LANES)]
                          for gr in range(4)]
                    lo = [plsc.bitcast(w << 16, jnp.float32) for w in ws]
                    hi = [plsc.bitcast(w & jnp.int32(-65536), jnp.float32)
                          for w in ws]
                    for which, (a, b, cc, dd) in ((0, lo), (1, hi)):
                        inner = cc + dd * frac
                        inner = b + inner * frac
                        res[which].append(a + inner * frac)
                for which in range(2):
                    s = 2 * p + which
                    tr = transpose16(res[which])
                    for r in range(LANES):
                        out_v[s * LANES + r, pl.ds(g * LANES, LANES)] = tr[r]

    pltpu.async_copy(coeffs_hbm.at[idx_v.at[0]], rows_b[0], gsem[0])

    def pair_body(jj, _):
        for b in range(2):
            j = jj * 2 + b

            @pl.when(j + 1 < NCH)
            def _():
                pltpu.async_copy(coeffs_hbm.at[idx_v.at[j + 1]],
                                 rows_b[1 - b], gsem[1 - b])

            pltpu.make_async_copy(coeffs_hbm.at[idx_v.at[j]],
                                  rows_b[b], gsem[b]).wait()

            @pl.when(j >= 2)
            def _():
                pltpu.make_async_copy(
                    out_b[b], out_hbm.at[:, pl.ds(base + (j - 2) * G, G)],
                    osem[b]).wait()

            compute_chunk(j, rows_b[b], out_b[b])
            pltpu.async_copy(out_b[b], out_hbm.at[:, pl.ds(base + j * G, G)],
                             osem[b])
        return 0

    lax.fori_loop(0, NCH // 2, pair_body, 0)

    for b in range(2):
        pltpu.make_async_copy(
            out_b[b], out_hbm.at[:, pl.ds(base + (NCH - 2 + b) * G, G)],
            osem[b]).wait()


@jax.jit
def _spline_sc(t_part, packed):
    mesh = plsc.VectorSubcoreMesh(core_axis_name="c", subcore_axis_name="s")
    return pl.kernel(
        _spline_body,
        mesh=mesh,
        compiler_params=pltpu.CompilerParams(needs_layout_passes=False),
        out_type=jax.ShapeDtypeStruct((C, NK), jnp.float32),
        scratch_types=[
            pltpu.VMEM((NQ,), jnp.float32),
            pltpu.VMEM((NCH, G), jnp.int32),
            pltpu.VMEM((G, 2 * C), jnp.int32),
            pltpu.VMEM((G, 2 * C), jnp.int32),
            pltpu.VMEM((C, G), jnp.float32),
            pltpu.VMEM((C, G), jnp.float32),
            pltpu.SemaphoreType.DMA,
            pltpu.SemaphoreType.DMA,
            pltpu.SemaphoreType.DMA,
            pltpu.SemaphoreType.DMA,
        ],
    )(t_part, packed)


def kernel(t, coeffs, t_grid):
    del t_grid
    scale = jnp.concatenate([jnp.full((2 * C,), 1.0, jnp.float32),
                             jnp.full((C,), 0.5, jnp.float32),
                             jnp.full((C,), 1.0 / 3.0, jnp.float32)])
    cb = (coeffs * scale).astype(jnp.bfloat16)
    x = cb.reshape(L_GRID, 8, 2, LANES).transpose(0, 1, 3, 2)
    u16 = lax.bitcast_convert_type(x, jnp.uint16)
    packed = lax.bitcast_convert_type(u16, jnp.int32).reshape(L_GRID, 2 * C)
    return _spline_sc(t, packed).T


# speedup vs baseline: 2.7437x; 1.0840x over previous
import functools

import jax
import jax.numpy as jnp
from jax import lax
from jax.experimental import pallas as pl
from jax.experimental.pallas import tpu as pltpu
from jax.experimental.pallas import tpu_sc as plsc

L_GRID = 8192
C = 64
N = 524288
LANES = 16
NW = 32
K = 1
NK = N // K
NQ = NK // NW
G = 128
NCH = NQ // G
GROUPS = G // LANES


def _spline_body(t_hbm, coeffs_hbm, out_hbm, t_v, idx_v,
                 rows_v0, rows_v1, out_v0, out_v1,
                 gsem0, gsem1, osem0, osem1):
    rows_b = (rows_v0, rows_v1)
    out_b = (out_v0, out_v1)
    gsem = (gsem0, gsem1)
    osem = (osem0, osem1)
    wid = lax.axis_index("s") * 2 + lax.axis_index("c")
    base = wid * NQ

    pltpu.sync_copy(t_hbm.at[pl.ds(base, NQ)], t_v)

    def idx_body(g, _):
        tv = t_v[pl.ds(g * LANES, LANES)]
        ti = tv.astype(jnp.int32)
        tf = ti.astype(jnp.float32)
        ceil_m1 = ti + jnp.where(tv > tf, 0, -1)
        idx = jnp.minimum(jnp.maximum(ceil_m1, 0), L_GRID - 1)
        frac = tv - idx.astype(jnp.float32)
        row = g // GROUPS
        col = (g % GROUPS) * LANES
        idx_v[row, pl.ds(col, LANES)] = idx
        t_v[pl.ds(g * LANES, LANES)] = frac
        return 0

    lax.fori_loop(0, NQ // LANES, idx_body, 0)

    lane_iota = lax.iota(jnp.int32, LANES)

    def transpose16(vs):
        cur = list(vs)
        for d in (1, 2, 4, 8):
            msk = (lane_iota & d) == 0
            pm = (lane_iota - d) & (LANES - 1)
            pp = (lane_iota + d) & (LANES - 1)
            nxt = []
            for i in range(LANES):
                part = cur[i ^ d]
                if (i & d) == 0:
                    sh = jnp.take_along_axis(part, pm, axis=0)
                    nxt.append(jnp.where(msk, cur[i], sh))
                else:
                    sh = jnp.take_along_axis(part, pp, axis=0)
                    nxt.append(jnp.where(msk, sh, cur[i]))
            cur = nxt
        return cur

    def compute_chunk(j, rows_v, out_v):
        @plsc.parallel_loop(0, GROUPS, unroll=1)
        def g_body(g):
            frac16 = t_v[pl.ds(j * G + g * LANES, LANES)]
            for p in range(2):
                res = ([], [])
                for i in range(LANES):
                    q = g * LANES + i
                    frac = jnp.full((LANES,), frac16[i])
                    ws = [rows_v[q, pl.ds((2 * gr + p) * LANES, LANES)]
                          for gr in range(4)]
                    lo = [plsc.bitcast(w << 16, jnp.float32) for w in ws]
                    hi = [plsc.bitcast(w, jnp.float32) for w in ws]
                    for which, (a, b, cc, dd) in ((0, lo), (1, hi)):
                        inner = cc + dd * frac
                        inner = b + inner * frac
                        res[which].append(a + inner * frac)
                for which in range(2):
                    s = 2 * p + which
                    tr = transpose16(res[which])
                    for r in range(LANES):
                        out_v[s * LANES + r, pl.ds(g * LANES, LANES)] = tr[r]

    pltpu.async_copy(coeffs_hbm.at[idx_v.at[0]], rows_b[0], gsem[0])

    def pair_body(jj, _):
        for b in range(2):
            j = jj * 2 + b

            @pl.when(j + 1 < NCH)
            def _():
                pltpu.async_copy(coeffs_hbm.at[idx_v.at[j + 1]],
                                 rows_b[1 - b], gsem[1 - b])

            pltpu.make_async_copy(coeffs_hbm.at[idx_v.at[j]],
                                  rows_b[b], gsem[b]).wait()

            @pl.when(j >= 2)
            def _():
                pltpu.make_async_copy(
                    out_b[b], out_hbm.at[:, pl.ds(base + (j - 2) * G, G)],
                    osem[b]).wait()

            compute_chunk(j, rows_b[b], out_b[b])
            pltpu.async_copy(out_b[b], out_hbm.at[:, pl.ds(base + j * G, G)],
                             osem[b])
        return 0

    lax.fori_loop(0, NCH // 2, pair_body, 0)

    for b in range(2):
        pltpu.make_async_copy(
            out_b[b], out_hbm.at[:, pl.ds(base + (NCH - 2 + b) * G, G)],
            osem[b]).wait()


@jax.jit
def _spline_sc(t_part, packed):
    mesh = plsc.VectorSubcoreMesh(core_axis_name="c", subcore_axis_name="s")
    return pl.kernel(
        _spline_body,
        mesh=mesh,
        compiler_params=pltpu.CompilerParams(needs_layout_passes=False),
        out_type=jax.ShapeDtypeStruct((C, NK), jnp.float32),
        scratch_types=[
            pltpu.VMEM((NQ,), jnp.float32),
            pltpu.VMEM((NCH, G), jnp.int32),
            pltpu.VMEM((G, 2 * C), jnp.int32),
            pltpu.VMEM((G, 2 * C), jnp.int32),
            pltpu.VMEM((C, G), jnp.float32),
            pltpu.VMEM((C, G), jnp.float32),
            pltpu.SemaphoreType.DMA,
            pltpu.SemaphoreType.DMA,
            pltpu.SemaphoreType.DMA,
            pltpu.SemaphoreType.DMA,
        ],
    )(t_part, packed)


def kernel(t, coeffs, t_grid):
    del t_grid
    scale = jnp.concatenate([jnp.full((2 * C,), 1.0, jnp.float32),
                             jnp.full((C,), 0.5, jnp.float32),
                             jnp.full((C,), 1.0 / 3.0, jnp.float32)])
    cb = (coeffs * scale).astype(jnp.bfloat16)
    x = cb.reshape(L_GRID, 8, 2, LANES).transpose(0, 1, 3, 2)
    u16 = lax.bitcast_convert_type(x, jnp.uint16)
    packed = lax.bitcast_convert_type(u16, jnp.int32).reshape(L_GRID, 2 * C)
    return _spline_sc(t, packed).T
